# baseline (speedup 1.0000x reference)
import functools

import jax
import jax.numpy as jnp
from jax import lax
from jax.experimental import pallas as pl
from jax.experimental.pallas import tpu as pltpu
from jax.experimental.pallas import tpu_sc as plsc

N = 10000
E = 160000
D = 364
OUT = 150
B = 4096

N_PAD = 10240
D_PAD = 384
HALF = 192
QUAR = 96
E_PAD = 163840
EPT = E_PAD // 16
ECHUNK = 128
NRING = 5
RPT = N_PAD // 16
OUT_PAD = 256
BB = 2 * B
VPT = BB // 16

_SC_MESH = dict(core_axis_name="c", subcore_axis_name="s")


def _leaky(x):
    return jnp.where(x > 0, x, 0.01 * x)


def _sc_segsum_body(xq, src_hbm, dst_hbm, out_hbm,
                    sidx, gidx, didx, gbuf, acc, isem, gsem, ssem):
    c = lax.axis_index("c")
    s = lax.axis_index("s")
    iota = lax.iota(jnp.int32, 16)

    ebase = s * EPT
    rbase = s * RPT
    nchunks = EPT // ECHUNK

    for phase in range(2):
        q = 2 * phase + c

        for i in range(RPT // ECHUNK):
            b = i % 2
            for j in range(8):
                rows = iota + (rbase + i * ECHUNK + j * 16)
                gidx[b, pl.ds(j * 16, 16)] = rows * 4 + q
            pltpu.async_copy(xq.at[gidx.at[b]], gbuf.at[b],
                             gsem.at[b]).wait()
            pltpu.sync_copy(gbuf.at[b],
                            acc.at[pl.ds(rbase + i * ECHUNK, ECHUNK)])
        plsc.subcore_barrier()

        def idx_load(i, t):
            eo = ebase + i * ECHUNK
            pltpu.async_copy(src_hbm.at[pl.ds(eo, ECHUNK)], sidx.at[t],
                             isem.at[t])
            pltpu.async_copy(dst_hbm.at[pl.ds(eo, ECHUNK)], didx.at[t],
                             isem.at[t])

        def idx_wait(i, t):
            eo = ebase + i * ECHUNK
            pltpu.make_async_copy(src_hbm.at[pl.ds(eo, ECHUNK)],
                                  sidx.at[t], isem.at[t]).wait()
            pltpu.make_async_copy(dst_hbm.at[pl.ds(eo, ECHUNK)],
                                  didx.at[t], isem.at[t]).wait()

        def make_gidx(t):
            for j in range(8):
                gidx[t, pl.ds(j * 16, 16)] = (
                    sidx[t, pl.ds(j * 16, 16)] * 4 + q)

        def gather(t):
            pltpu.async_copy(xq.at[gidx.at[t]], gbuf.at[t], gsem.at[t])

        def gather_wait(t):
            pltpu.make_async_copy(
                xq.at[gidx.at[t]], gbuf.at[t], gsem.at[t]).wait()

        def scatter(t):
            pltpu.async_copy(gbuf.at[t], acc.at[didx.at[t]], ssem.at[t],
                             add=True)

        def scatter_wait(t):
            pltpu.make_async_copy(gbuf.at[t], acc.at[didx.at[t]],
                                  ssem.at[t]).wait()

        idx_load(0, 0)
        idx_load(1, 1)
        idx_wait(0, 0)
        make_gidx(0)
        gather(0)

        def group(g, _):
            i0 = NRING * g
            for t in range(NRING):
                i = i0 + t
                t1 = (t + 1) % NRING
                t2 = (t + 2) % NRING

                @pl.when(i + 1 < nchunks)
                def _():
                    idx_wait(i + 1, t1)
                    make_gidx(t1)
                    gather(t1)

                @pl.when(i + 2 < nchunks)
                def _():
                    @pl.when(i >= NRING - 2)
                    def _():
                        scatter_wait(t2)
                    idx_load(i + 2, t2)
                gather_wait(t)
                scatter(t)
            return 0

        lax.fori_loop(0, nchunks // NRING, group, 0)
        for t in range(NRING):
            scatter_wait(t)
        plsc.subcore_barrier()

        for i in range(RPT // ECHUNK):
            b = i % 2
            pltpu.sync_copy(acc.at[pl.ds(rbase + i * ECHUNK, ECHUNK)],
                            gbuf.at[b])
            for j in range(8):
                rows = iota + (rbase + i * ECHUNK + j * 16)
                gidx[b, pl.ds(j * 16, 16)] = rows * 4 + q
            pltpu.async_copy(gbuf.at[b], out_hbm.at[gidx.at[b]],
                             gsem.at[b]).wait()


@functools.cache
def _sc_segsum():
    return pl.kernel(
        _sc_segsum_body,
        mesh=plsc.VectorSubcoreMesh(**_SC_MESH),
        compiler_params=pltpu.CompilerParams(use_tc_tiling_on_sc=False),
        out_type=jax.ShapeDtypeStruct((4 * N_PAD, QUAR), jnp.float32),
        scratch_types=[
            pltpu.VMEM((NRING, ECHUNK), jnp.int32),
            pltpu.VMEM((NRING, ECHUNK), jnp.int32),
            pltpu.VMEM((NRING, ECHUNK), jnp.int32),
            pltpu.VMEM((NRING, ECHUNK, QUAR), jnp.float32),
            pltpu.VMEM_SHARED((N_PAD, QUAR), jnp.float32),
            pltpu.SemaphoreType.DMA((NRING,)),
            pltpu.SemaphoreType.DMA((NRING,)),
            pltpu.SemaphoreType.DMA((NRING,)),
        ],
    )


def _sc_gather_body(s2r, vcat_hbm, out_hbm, vcat_v, gidx, oidx, gbuf, sem):
    c = lax.axis_index("c")
    s = lax.axis_index("s")
    iota = lax.iota(jnp.int32, 16)
    vbase = s * VPT
    pltpu.sync_copy(vcat_hbm.at[pl.ds(vbase, VPT)], vcat_v)
    for k in range(VPT // ECHUNK):
        for j in range(8):
            off = k * ECHUNK + j * 16
            vv = vcat_v[pl.ds(off, 16)]
            gidx[pl.ds(j * 16, 16)] = vv * 2 + c
            oidx[pl.ds(j * 16, 16)] = (iota + vbase + off) * 2 + c
        pltpu.async_copy(s2r.at[gidx], gbuf, sem).wait()
        pltpu.async_copy(gbuf, out_hbm.at[oidx], sem).wait()


@functools.cache
def _sc_gather():
    return pl.kernel(
        _sc_gather_body,
        mesh=plsc.VectorSubcoreMesh(**_SC_MESH),
        compiler_params=pltpu.CompilerParams(use_tc_tiling_on_sc=False),
        out_type=jax.ShapeDtypeStruct((2 * BB, HALF), jnp.float32),
        scratch_types=[
            pltpu.VMEM((VPT,), jnp.int32),
            pltpu.VMEM((ECHUNK,), jnp.int32),
            pltpu.VMEM((ECHUNK,), jnp.int32),
            pltpu.VMEM((ECHUNK, HALF), jnp.float32),
            pltpu.SemaphoreType.DMA,
        ],
    )


def _tc_linear_body(s_ref, w_ref, b_ref, o_ref):
    acc = jnp.dot(s_ref[...], w_ref[...], preferred_element_type=jnp.float32)
    o_ref[...] = _leaky(acc + b_ref[...])


def _tc_linear(sarr, wt, b):
    n = sarr.shape[0]
    blk = 512
    return pl.pallas_call(
        _tc_linear_body,
        grid=(n // blk,),
        in_specs=[
            pl.BlockSpec((blk, D_PAD), lambda i: (i, 0)),
            pl.BlockSpec((D_PAD, D_PAD), lambda i: (0, 0)),
            pl.BlockSpec((1, D_PAD), lambda i: (0, 0)),
        ],
        out_specs=pl.BlockSpec((blk, D_PAD), lambda i: (i, 0)),
        out_shape=jax.ShapeDtypeStruct((n, D_PAD), jnp.float32),
    )(sarr, wt, b)


def _tc_final_body(u_ref, w2_ref, b2_ref, w3_ref, b3_ref, o_ref):
    t = jnp.dot(u_ref[...], w2_ref[...], preferred_element_type=jnp.float32)
    t = _leaky(t + b2_ref[...])
    z = jnp.dot(t, w3_ref[...], preferred_element_type=jnp.float32)
    z = _leaky(z + b3_ref[...])
    n = jnp.sqrt(jnp.sum(z * z, axis=1, keepdims=True))
    o_ref[...] = z / jnp.maximum(n, 1e-12)


def _tc_final(u, w2t, b2, w3t, b3):
    blk = 512
    return pl.pallas_call(
        _tc_final_body,
        grid=(BB // blk,),
        in_specs=[
            pl.BlockSpec((blk, D_PAD), lambda i: (i, 0)),
            pl.BlockSpec((D_PAD, D_PAD), lambda i: (0, 0)),
            pl.BlockSpec((1, D_PAD), lambda i: (0, 0)),
            pl.BlockSpec((D_PAD, OUT_PAD), lambda i: (0, 0)),
            pl.BlockSpec((1, OUT_PAD), lambda i: (0, 0)),
        ],
        out_specs=pl.BlockSpec((blk, OUT_PAD), lambda i: (i, 0)),
        out_shape=jax.ShapeDtypeStruct((BB, OUT_PAD), jnp.float32),
    )(u, w2t, b2, w3t, b3)


def kernel(features, edge_index, v1, v2, W1, b1, W2, b2, W3, b3):
    xp = jnp.pad(features, ((0, N_PAD - N), (0, D_PAD - D)))
    src = jnp.pad(edge_index[0], (0, E_PAD - E))
    dst = jnp.pad(edge_index[1], (0, E_PAD - E), constant_values=N_PAD - 1)
    vcat = jnp.concatenate([v1, v2])

    w1t = jnp.pad(W1, ((0, D_PAD - D), (0, D_PAD - D))).T
    b1p = jnp.pad(b1, (0, D_PAD - D)).reshape(1, D_PAD)
    w2t = jnp.pad(W2, ((0, D_PAD - D), (0, D_PAD - D))).T
    b2p = jnp.pad(b2, (0, D_PAD - D)).reshape(1, D_PAD)
    w3t = jnp.pad(W3, ((0, OUT_PAD - OUT), (0, D_PAD - D))).T
    b3p = jnp.pad(b3, (0, OUT_PAD - OUT)).reshape(1, OUT_PAD)

    s1q = _sc_segsum()(xp.reshape(4 * N_PAD, QUAR), src, dst)
    h = _tc_linear(s1q.reshape(N_PAD, D_PAD), w1t, b1p)
    s2q = _sc_segsum()(h.reshape(4 * N_PAD, QUAR), src, dst)
    ur = _sc_gather()(s2q.reshape(2 * N_PAD, HALF), vcat)
    z = _tc_final(ur.reshape(BB, D_PAD), w2t, b2p, w3t, b3p)
    return (z[:B, :OUT], z[B:, :OUT])


# speedup vs baseline: 1.0171x; 1.0171x over previous
import functools

import jax
import jax.numpy as jnp
from jax import lax
from jax.experimental import pallas as pl
from jax.experimental.pallas import tpu as pltpu
from jax.experimental.pallas import tpu_sc as plsc

N = 10000
E = 160000
D = 364
OUT = 150
B = 4096

N_PAD = 10240
D_PAD = 384
HALF = 192
QUAR = 96
E_PAD = 163840
EPT = E_PAD // 16
ECHUNK = 128
RPT = N_PAD // 16
OUT_PAD = 256
BB = 2 * B
VPT = BB // 16

_SC_MESH = dict(core_axis_name="c", subcore_axis_name="s")


def _leaky(x):
    return jnp.where(x > 0, x, 0.01 * x)


def _sc_segsum_body(xq, src_hbm, dst_hbm, out_hbm,
                    sidx, gidx, didx, gbuf, acc, isem, gsem, ssem):
    c = lax.axis_index("c")
    s = lax.axis_index("s")
    iota = lax.iota(jnp.int32, 16)

    ebase = s * EPT
    rbase = s * RPT
    nchunks = EPT // ECHUNK

    nblk = RPT // ECHUNK

    def own_rows(i):
        return pl.ds(rbase + i * ECHUNK, ECHUNK)

    for phase in range(2):
        q = 2 * phase + c

        def ibuild(i, t):
            for j in range(8):
                rows = iota + (rbase + i * ECHUNK + j * 16)
                gidx[t, pl.ds(j * 16, 16)] = rows * 4 + q

        def igather(t):
            pltpu.async_copy(xq.at[gidx.at[t]], gbuf.at[t], gsem.at[t])

        def igwait(t):
            pltpu.make_async_copy(
                xq.at[gidx.at[t]], gbuf.at[t], gsem.at[t]).wait()

        ibuild(0, 0)
        igather(0)
        ibuild(1, 1)
        igather(1)
        for i in range(nblk):
            t = i % 4
            igwait(t)
            pltpu.async_copy(gbuf.at[t], acc.at[own_rows(i)], ssem.at[t])
            if i + 2 < nblk:
                tn = (i + 2) % 4
                if i + 2 >= 4:
                    pltpu.make_async_copy(gbuf.at[tn],
                                          acc.at[own_rows(i - 2)],
                                          ssem.at[tn]).wait()
                ibuild(i + 2, tn)
                igather(tn)
        for i in range(max(0, nblk - 4), nblk):
            t = i % 4
            pltpu.make_async_copy(gbuf.at[t], acc.at[own_rows(i)],
                                  ssem.at[t]).wait()
        plsc.subcore_barrier()

        def idx_load(i, t):
            eo = ebase + i * ECHUNK
            pltpu.async_copy(src_hbm.at[pl.ds(eo, ECHUNK)], sidx.at[t],
                             isem.at[t])
            pltpu.async_copy(dst_hbm.at[pl.ds(eo, ECHUNK)], didx.at[t],
                             isem.at[t])

        def idx_wait(i, t):
            eo = ebase + i * ECHUNK
            pltpu.make_async_copy(src_hbm.at[pl.ds(eo, ECHUNK)],
                                  sidx.at[t], isem.at[t]).wait()
            pltpu.make_async_copy(dst_hbm.at[pl.ds(eo, ECHUNK)],
                                  didx.at[t], isem.at[t]).wait()

        def make_gidx(t):
            for j in range(8):
                gidx[t, pl.ds(j * 16, 16)] = (
                    sidx[t, pl.ds(j * 16, 16)] * 4 + q)

        def gather(t):
            pltpu.async_copy(xq.at[gidx.at[t]], gbuf.at[t], gsem.at[t])

        def gather_wait(t):
            pltpu.make_async_copy(
                xq.at[gidx.at[t]], gbuf.at[t], gsem.at[t]).wait()

        def scatter(t):
            pltpu.async_copy(gbuf.at[t], acc.at[didx.at[t]], ssem.at[t],
                             add=True)

        def scatter_wait(t):
            pltpu.make_async_copy(gbuf.at[t], acc.at[didx.at[t]],
                                  ssem.at[t]).wait()

        idx_load(0, 0)
        idx_load(1, 1)
        idx_wait(0, 0)
        make_gidx(0)
        gather(0)

        def group(g, _):
            i0 = 4 * g
            for t in range(4):
                i = i0 + t
                t1 = (t + 1) % 4
                t2 = (t + 2) % 4

                @pl.when(i + 1 < nchunks)
                def _():
                    idx_wait(i + 1, t1)
                    make_gidx(t1)
                    gather(t1)

                @pl.when(i + 2 < nchunks)
                def _():
                    @pl.when(i >= 2)
                    def _():
                        scatter_wait(t2)
                    idx_load(i + 2, t2)
                gather_wait(t)
                scatter(t)
            return 0

        lax.fori_loop(0, nchunks // 4, group, 0)
        for t in range(4):
            scatter_wait(t)
        plsc.subcore_barrier()

        def wload(i, t):
            pltpu.async_copy(acc.at[own_rows(i)], gbuf.at[t], ssem.at[t])

        def wload_wait(i, t):
            pltpu.make_async_copy(acc.at[own_rows(i)], gbuf.at[t],
                                  ssem.at[t]).wait()

        def wscat(t):
            pltpu.async_copy(gbuf.at[t], out_hbm.at[gidx.at[t]],
                             gsem.at[t])

        def wscat_wait(t):
            pltpu.make_async_copy(gbuf.at[t], out_hbm.at[gidx.at[t]],
                                  gsem.at[t]).wait()

        wload(0, 0)
        wload(1, 1)
        for i in range(nblk):
            t = i % 4
            wload_wait(i, t)
            ibuild(i, t)
            wscat(t)
            if i + 2 < nblk:
                tn = (i + 2) % 4
                if i + 2 >= 4:
                    wscat_wait(tn)
                wload(i + 2, tn)
        for i in range(max(0, nblk - 4), nblk):
            wscat_wait(i % 4)


@functools.cache
def _sc_segsum():
    return pl.kernel(
        _sc_segsum_body,
        mesh=plsc.VectorSubcoreMesh(**_SC_MESH),
        compiler_params=pltpu.CompilerParams(use_tc_tiling_on_sc=False),
        out_type=jax.ShapeDtypeStruct((4 * N_PAD, QUAR), jnp.float32),
        scratch_types=[
            pltpu.VMEM((4, ECHUNK), jnp.int32),
            pltpu.VMEM((4, ECHUNK), jnp.int32),
            pltpu.VMEM((4, ECHUNK), jnp.int32),
            pltpu.VMEM((4, ECHUNK, QUAR), jnp.float32),
            pltpu.VMEM_SHARED((N_PAD, QUAR), jnp.float32),
            pltpu.SemaphoreType.DMA((4,)),
            pltpu.SemaphoreType.DMA((4,)),
            pltpu.SemaphoreType.DMA((4,)),
        ],
    )


def _sc_gather_body(s2r, vcat_hbm, out_hbm, vcat_v, gidx, oidx, gbuf, sem):
    c = lax.axis_index("c")
    s = lax.axis_index("s")
    iota = lax.iota(jnp.int32, 16)
    vbase = s * VPT
    pltpu.sync_copy(vcat_hbm.at[pl.ds(vbase, VPT)], vcat_v)
    for k in range(VPT // ECHUNK):
        for j in range(8):
            off = k * ECHUNK + j * 16
            vv = vcat_v[pl.ds(off, 16)]
            gidx[pl.ds(j * 16, 16)] = vv * 2 + c
            oidx[pl.ds(j * 16, 16)] = (iota + vbase + off) * 2 + c
        pltpu.async_copy(s2r.at[gidx], gbuf, sem).wait()
        pltpu.async_copy(gbuf, out_hbm.at[oidx], sem).wait()


@functools.cache
def _sc_gather():
    return pl.kernel(
        _sc_gather_body,
        mesh=plsc.VectorSubcoreMesh(**_SC_MESH),
        compiler_params=pltpu.CompilerParams(use_tc_tiling_on_sc=False),
        out_type=jax.ShapeDtypeStruct((2 * BB, HALF), jnp.float32),
        scratch_types=[
            pltpu.VMEM((VPT,), jnp.int32),
            pltpu.VMEM((ECHUNK,), jnp.int32),
            pltpu.VMEM((ECHUNK,), jnp.int32),
            pltpu.VMEM((ECHUNK, HALF), jnp.float32),
            pltpu.SemaphoreType.DMA,
        ],
    )


def _tc_linear_body(s_ref, w_ref, b_ref, o_ref):
    acc = jnp.dot(s_ref[...], w_ref[...], preferred_element_type=jnp.float32)
    o_ref[...] = _leaky(acc + b_ref[...])


def _tc_linear(sarr, wt, b):
    n = sarr.shape[0]
    blk = 512
    return pl.pallas_call(
        _tc_linear_body,
        grid=(n // blk,),
        in_specs=[
            pl.BlockSpec((blk, D_PAD), lambda i: (i, 0)),
            pl.BlockSpec((D_PAD, D_PAD), lambda i: (0, 0)),
            pl.BlockSpec((1, D_PAD), lambda i: (0, 0)),
        ],
        out_specs=pl.BlockSpec((blk, D_PAD), lambda i: (i, 0)),
        out_shape=jax.ShapeDtypeStruct((n, D_PAD), jnp.float32),
    )(sarr, wt, b)


def _tc_final_body(u_ref, w2_ref, b2_ref, w3_ref, b3_ref, o_ref):
    t = jnp.dot(u_ref[...], w2_ref[...], preferred_element_type=jnp.float32)
    t = _leaky(t + b2_ref[...])
    z = jnp.dot(t, w3_ref[...], preferred_element_type=jnp.float32)
    z = _leaky(z + b3_ref[...])
    n = jnp.sqrt(jnp.sum(z * z, axis=1, keepdims=True))
    o_ref[...] = z / jnp.maximum(n, 1e-12)


def _tc_final(u, w2t, b2, w3t, b3):
    blk = 512
    return pl.pallas_call(
        _tc_final_body,
        grid=(BB // blk,),
        in_specs=[
            pl.BlockSpec((blk, D_PAD), lambda i: (i, 0)),
            pl.BlockSpec((D_PAD, D_PAD), lambda i: (0, 0)),
            pl.BlockSpec((1, D_PAD), lambda i: (0, 0)),
            pl.BlockSpec((D_PAD, OUT_PAD), lambda i: (0, 0)),
            pl.BlockSpec((1, OUT_PAD), lambda i: (0, 0)),
        ],
        out_specs=pl.BlockSpec((blk, OUT_PAD), lambda i: (i, 0)),
        out_shape=jax.ShapeDtypeStruct((BB, OUT_PAD), jnp.float32),
    )(u, w2t, b2, w3t, b3)


def kernel(features, edge_index, v1, v2, W1, b1, W2, b2, W3, b3):
    xp = jnp.pad(features, ((0, N_PAD - N), (0, D_PAD - D)))
    src = jnp.pad(edge_index[0], (0, E_PAD - E))
    dst = jnp.pad(edge_index[1], (0, E_PAD - E), constant_values=N_PAD - 1)
    vcat = jnp.concatenate([v1, v2])

    w1t = jnp.pad(W1, ((0, D_PAD - D), (0, D_PAD - D))).T
    b1p = jnp.pad(b1, (0, D_PAD - D)).reshape(1, D_PAD)
    w2t = jnp.pad(W2, ((0, D_PAD - D), (0, D_PAD - D))).T
    b2p = jnp.pad(b2, (0, D_PAD - D)).reshape(1, D_PAD)
    w3t = jnp.pad(W3, ((0, OUT_PAD - OUT), (0, D_PAD - D))).T
    b3p = jnp.pad(b3, (0, OUT_PAD - OUT)).reshape(1, OUT_PAD)

    s1q = _sc_segsum()(xp.reshape(4 * N_PAD, QUAR), src, dst)
    h = _tc_linear(s1q.reshape(N_PAD, D_PAD), w1t, b1p)
    s2q = _sc_segsum()(h.reshape(4 * N_PAD, QUAR), src, dst)
    ur = _sc_gather()(s2q.reshape(2 * N_PAD, HALF), vcat)
    z = _tc_final(ur.reshape(BB, D_PAD), w2t, b2p, w3t, b3p)
    return (z[:B, :OUT], z[B:, :OUT])
